# Initial kernel scaffold; baseline (speedup 1.0000x reference)
import jax
import jax.numpy as jnp
from jax.experimental import pallas as pl


def kernel(points, point_features, Wp1, bp1, Wp2, bp2, Wp3, bp3, Wf1, bf1, Wf2, bf2, Wf3, bf3):
    raise NotImplementedError("write your pallas kernel here")


# speedup vs baseline: 11.1423x; 11.1423x over previous
import functools

import jax
import jax.numpy as jnp
import numpy as np
from jax import lax
from jax.experimental import pallas as pl
from jax.experimental.pallas import tpu as pltpu
from jax.experimental.pallas import tpu_sc as plsc

B = 8
N = 4096
CIN = 64
COUT = 128
K = N // 4
GROUP = 64
RAD2 = np.float32(0.2 ** 2)
NC, NS = 2, 16
NW = NC * NS
ROWS = B * K
RPW = ROWS // NW

_MAXI32 = np.int32(0x7FFFFFFF)
_RAD2_BITS = int(np.float32(0.2 ** 2).view(np.int32))


def _fps_body(pts_ref, cx_ref, cy_ref, cz_ref):
    X = pts_ref[:, 0, :]
    Y = pts_ref[:, 1, :]
    Z = pts_ref[:, 2, :]
    iota = lax.broadcasted_iota(jnp.int32, (B, N), 1)

    def extract(V, last):
        m = (iota == last).astype(jnp.float32)
        return jnp.sum(V * m, axis=1, keepdims=True)

    def step(t, carry):
        dists, last = carry
        cx = extract(X, last)
        cy = extract(Y, last)
        cz = extract(Z, last)
        cx_ref[pl.ds(t, 1), :] = cx.T
        cy_ref[pl.ds(t, 1), :] = cy.T
        cz_ref[pl.ds(t, 1), :] = cz.T
        dx = X - cx
        dy = Y - cy
        dz = Z - cz
        d = (dx * dx + dy * dy) + dz * dz
        dists = jnp.minimum(dists, d)
        mx = jnp.max(dists, axis=1, keepdims=True)
        nxt = jnp.min(jnp.where(dists == mx, iota, N), axis=1, keepdims=True)
        return dists, nxt

    init = (jnp.full((B, N), jnp.inf, jnp.float32), jnp.zeros((B, 1), jnp.int32))
    lax.fori_loop(0, K, step, init)


def _fps(points):
    out = pl.pallas_call(
        _fps_body,
        out_shape=[jax.ShapeDtypeStruct((K, B), jnp.float32)] * 3,
    )(points)
    return out


_KT = 256


def _d2_body(pts_ref, cx_ref, cy_ref, cz_ref, out_ref):
    X = pts_ref[0, 0, :][None, :]
    Y = pts_ref[0, 1, :][None, :]
    Z = pts_ref[0, 2, :][None, :]
    cx = cx_ref[0, 0, :][:, None]
    cy = cy_ref[0, 0, :][:, None]
    cz = cz_ref[0, 0, :][:, None]
    dx = cx - X
    dy = cy - Y
    dz = cz - Z
    out_ref[0] = (dx * dx + dy * dy) + dz * dz


def _d2(points, cx, cy, cz):
    grid = (B, K // _KT)
    cspec = pl.BlockSpec((1, 1, _KT), lambda b, k: (b, 0, k))
    return pl.pallas_call(
        _d2_body,
        grid=grid,
        in_specs=[
            pl.BlockSpec((1, 3, N), lambda b, k: (b, 0, 0)),
            cspec, cspec, cspec,
        ],
        out_specs=pl.BlockSpec((1, _KT, N), lambda b, k: (b, k, 0)),
        out_shape=jax.ShapeDtypeStruct((B, K, N), jnp.float32),
    )(points, cx, cy, cz)


def _sel_body(d_hbm, gidx_hbm, dbuf, vbuf, ibuf, sbuf, obuf, sem):
    wid = lax.axis_index("s") * NC + lax.axis_index("c")
    base = wid * RPW
    lane = lax.iota(jnp.int32, 16)
    zeros16 = jnp.zeros((16,), jnp.int32)

    one16 = jnp.full((16,), 1, jnp.int32)
    radv = jnp.full((16,), RAD2, jnp.float32)

    def row_body(g, _):
        row = base + g
        pltpu.async_copy(d_hbm.at[pl.ds(row * N, N)], dbuf, sem).wait()

        def cbody(j, cnt):
            v = dbuf[pl.ds(j * 16, 16)]
            m = v <= radv
            mi = jnp.where(m, one16, zeros16)
            inc = plsc.cumsum(mi)
            pos = cnt + inc - mi
            plsc.store_scatter(vbuf, [pos], v, mask=m)
            plsc.store_scatter(ibuf, [pos], lane + j * 16, mask=m)
            return cnt + plsc.all_reduce_population_count(m)

        cnt_vec = lax.fori_loop(0, N // 16, cbody, zeros16)
        cnt = cnt_vec[0]
        nv = (cnt + 15) >> 4

        def count_le(mid):
            def kbody(j, c):
                kv = plsc.bitcast(vbuf[pl.ds(j * 16, 16)], jnp.int32)
                valid = (lane + j * 16) < cnt_vec
                return c + plsc.all_reduce_population_count((kv <= mid) & valid)

            return lax.fori_loop(0, nv, kbody, zeros16)

        def bbody(_, lohi):
            lo, hi = lohi
            mid = (lo + hi) >> 1
            c = count_le(mid)
            upd = lo < hi
            ge = c >= 64
            hi = jnp.where(upd & ge, mid, hi)
            lo = jnp.where(upd & (~ge), mid + 1, lo)
            return lo, hi

        lo0 = zeros16
        hi0 = jnp.full((16,), _RAD2_BITS + 1, jnp.int32)
        t_vec, _hi = lax.fori_loop(0, 31, bbody, (lo0, hi0))
        t_vec = jnp.where(cnt_vec <= 64, jnp.full((16,), _MAXI32), t_vec)

        pad = ibuf[pl.ds(0, 16)][0]
        for q in range(4):
            obuf[pl.ds(q * 16, 16)] = jnp.full((16,), pad, jnp.int32)

        def emit_lt(j, c_lt):
            kv = plsc.bitcast(vbuf[pl.ds(j * 16, 16)], jnp.int32)
            iv = ibuf[pl.ds(j * 16, 16)]
            valid = (lane + j * 16) < cnt_vec
            m_lt = (kv < t_vec) & valid
            mi = jnp.where(m_lt, one16, zeros16)
            pos = c_lt + plsc.cumsum(mi) - mi
            plsc.store_scatter(obuf, [pos], iv, mask=m_lt)
            return c_lt + plsc.all_reduce_population_count(m_lt)

        c_lt = lax.fori_loop(0, nv, emit_lt, zeros16)

        cap64 = jnp.full((16,), 64, jnp.int32)

        def emit_eq(j, c_eq):
            kv = plsc.bitcast(vbuf[pl.ds(j * 16, 16)], jnp.int32)
            iv = ibuf[pl.ds(j * 16, 16)]
            valid = (lane + j * 16) < cnt_vec
            m_eq = (kv == t_vec) & valid
            me = jnp.where(m_eq, one16, zeros16)
            pos_e = c_eq + plsc.cumsum(me) - me
            plsc.store_scatter(obuf, [pos_e], iv, mask=m_eq & (pos_e < cap64))
            return c_eq + plsc.all_reduce_population_count(m_eq)

        lax.fori_loop(0, nv, emit_eq, c_lt)
        pltpu.async_copy(obuf, gidx_hbm.at[pl.ds(row * GROUP, GROUP)], sem).wait()
        return 0

    lax.fori_loop(0, RPW, row_body, 0)


def _select(d2):
    f = pl.kernel(
        _sel_body,
        out_type=jax.ShapeDtypeStruct((ROWS * GROUP,), jnp.int32),
        mesh=plsc.VectorSubcoreMesh(core_axis_name="c", subcore_axis_name="s"),
        compiler_params=pltpu.CompilerParams(needs_layout_passes=False),
        scratch_types=[
            pltpu.VMEM((N,), jnp.float32),
            pltpu.VMEM((N,), jnp.float32),
            pltpu.VMEM((N,), jnp.int32),
            pltpu.VMEM((16,), jnp.int32),
            pltpu.VMEM((GROUP,), jnp.int32),
            pltpu.SemaphoreType.DMA,
        ],
    )
    return f(d2.reshape(ROWS * N))


_RT = 2048


def _pre_body(ft_ref, pt_ref, wf1, bf1, wf2, bf2, wf3, bf3, wp1, bp1, p_ref, a_ref):
    h = jnp.maximum(jnp.dot(ft_ref[...], wf1[...], preferred_element_type=jnp.float32) + bf1[...], 0.0)
    h = jnp.maximum(jnp.dot(h, wf2[...], preferred_element_type=jnp.float32) + bf2[...], 0.0)
    p_ref[...] = jnp.dot(h, wf3[...], preferred_element_type=jnp.float32) + bf3[...]
    a_ref[...] = jnp.dot(pt_ref[...], wp1[...], preferred_element_type=jnp.float32) + bp1[...]


def _precompute(ft, pt8, wf1t, bf1, wf2t, bf2, wf3t, bf3, wp1t8, bp1):
    grid = (B * N // _RT,)
    wspec = pl.BlockSpec((64, 64), lambda i: (0, 0))
    bspec = pl.BlockSpec((1, 64), lambda i: (0, 0))
    return pl.pallas_call(
        _pre_body,
        grid=grid,
        in_specs=[
            pl.BlockSpec((_RT, 64), lambda i: (i, 0)),
            pl.BlockSpec((_RT, 8), lambda i: (i, 0)),
            wspec, bspec, wspec, bspec, wspec, bspec,
            pl.BlockSpec((8, 64), lambda i: (0, 0)), bspec,
        ],
        out_specs=[
            pl.BlockSpec((_RT, 64), lambda i: (i, 0)),
            pl.BlockSpec((_RT, 64), lambda i: (i, 0)),
        ],
        out_shape=[
            jax.ShapeDtypeStruct((B * N, 64), jnp.float32),
            jax.ShapeDtypeStruct((B * N, 64), jnp.float32),
        ],
    )(ft, pt8, wf1t, bf1, wf2t, bf2, wf3t, bf3, wp1t8, bp1)


def _cmat_body(ctr_ref, wp1_ref, c_ref):
    c_ref[...] = jnp.dot(ctr_ref[...], wp1_ref[...], preferred_element_type=jnp.float32)


def _cmat(ctr8, wp1t8):
    return pl.pallas_call(
        _cmat_body,
        out_shape=jax.ShapeDtypeStruct((ROWS, 64), jnp.float32),
    )(ctr8, wp1t8)


def _gath_body(gidx_hbm, a_hbm, p_hbm, c_hbm, h1_hbm, ff_hbm,
               idxb, abuf, pbuf, cbuf, h1buf, ffbuf, sem, sem2):
    wid = lax.axis_index("s") * NC + lax.axis_index("c")
    base = wid * RPW
    neg_inf = jnp.full((16,), -jnp.inf, jnp.float32)

    def row_body(g, _):
        row = base + g
        pltpu.async_copy(gidx_hbm.at[pl.ds(row * GROUP, GROUP)], idxb, sem).wait()
        boff = (row >> 10) * N
        boffv = jnp.full((16,), boff, jnp.int32)
        for q in range(4):
            idxb[pl.ds(q * 16, 16)] = idxb[pl.ds(q * 16, 16)] + boffv
        cpa = pltpu.async_copy(a_hbm.at[idxb], abuf, sem)
        cpp = pltpu.async_copy(p_hbm.at[idxb], pbuf, sem2)
        cpa.wait()
        cpp.wait()
        pltpu.async_copy(c_hbm.at[pl.ds(row * 64, 64)], cbuf, sem).wait()

        def rbody(r, acc):
            out = []
            for q in range(4):
                a = abuf[r, pl.ds(q * 16, 16)]
                c = cbuf[pl.ds(q * 16, 16)]
                h1buf[pl.ds(r * 64 + q * 16, 16)] = jnp.maximum(a - c, 0.0)
                out.append(jnp.maximum(acc[q], pbuf[r, pl.ds(q * 16, 16)]))
            return tuple(out)

        acc = lax.fori_loop(0, GROUP, rbody, (neg_inf,) * 4)
        for q in range(4):
            ffbuf[pl.ds(q * 16, 16)] = acc[q]
        pltpu.async_copy(h1buf, h1_hbm.at[pl.ds(row * GROUP * 64, GROUP * 64)], sem).wait()
        pltpu.async_copy(ffbuf, ff_hbm.at[pl.ds(row * 64, 64)], sem).wait()
        return 0

    lax.fori_loop(0, RPW, row_body, 0)


def _gather_l1(gidx, a_rows, p_rows, c_rows):
    f = pl.kernel(
        _gath_body,
        out_type=[
            jax.ShapeDtypeStruct((ROWS * GROUP * 64,), jnp.float32),
            jax.ShapeDtypeStruct((ROWS * 64,), jnp.float32),
        ],
        mesh=plsc.VectorSubcoreMesh(core_axis_name="c", subcore_axis_name="s"),
        compiler_params=pltpu.CompilerParams(needs_layout_passes=False,
                                             use_tc_tiling_on_sc=False),
        scratch_types=[
            pltpu.VMEM((GROUP,), jnp.int32),
            pltpu.VMEM((GROUP, 64), jnp.float32),
            pltpu.VMEM((GROUP, 64), jnp.float32),
            pltpu.VMEM((64,), jnp.float32),
            pltpu.VMEM((GROUP * 64,), jnp.float32),
            pltpu.VMEM((64,), jnp.float32),
            pltpu.SemaphoreType.DMA,
            pltpu.SemaphoreType.DMA,
        ],
    )
    return f(gidx, a_rows, p_rows, c_rows.reshape(ROWS * 64))


_GT = 64


def _mlp2_body(h1_ref, wp2, bp2, wp3, bp3, out_ref):
    h = jnp.maximum(jnp.dot(h1_ref[...], wp2[...], preferred_element_type=jnp.float32) + bp2[...], 0.0)
    h = jnp.dot(h, wp3[...], preferred_element_type=jnp.float32) + bp3[...]
    out_ref[...] = jnp.max(h.reshape(_GT, GROUP, 64), axis=1)


def _mlp2(h1, wp2t, bp2, wp3t, bp3):
    grid = (ROWS // _GT,)
    wspec = pl.BlockSpec((64, 64), lambda i: (0, 0))
    bspec = pl.BlockSpec((1, 64), lambda i: (0, 0))
    return pl.pallas_call(
        _mlp2_body,
        grid=grid,
        in_specs=[
            pl.BlockSpec((_GT * GROUP, 64), lambda i: (i, 0)),
            wspec, bspec, wspec, bspec,
        ],
        out_specs=pl.BlockSpec((_GT, 64), lambda i: (i, 0)),
        out_shape=jax.ShapeDtypeStruct((ROWS, 64), jnp.float32),
    )(h1, wp2t, bp2, wp3t, bp3)


def kernel(points, point_features, Wp1, bp1, Wp2, bp2, Wp3, bp3,
           Wf1, bf1, Wf2, bf2, Wf3, bf3):
    ctrx, ctry, ctrz = _fps(points)
    cx, cy, cz = ctrx.T, ctry.T, ctrz.T
    d2 = _d2(points, cx[:, None, :], cy[:, None, :], cz[:, None, :])
    gidx = _select(d2)

    ft = point_features.transpose(0, 2, 1).reshape(B * N, CIN)
    pt = points.transpose(0, 2, 1).reshape(B * N, 3)
    pt8 = jnp.pad(pt, ((0, 0), (0, 5)))
    wp1t8 = jnp.pad(Wp1.T, ((0, 5), (0, 0)))
    p_rows, a_rows = _precompute(
        ft, pt8, Wf1.T, bf1[None, :], Wf2.T, bf2[None, :], Wf3.T, bf3[None, :],
        wp1t8, bp1[None, :])

    ctr_bk3 = jnp.stack([cx, cy, cz], axis=-1)
    ctr8 = jnp.pad(ctr_bk3.reshape(ROWS, 3), ((0, 0), (0, 5)))
    c_rows = _cmat(ctr8, wp1t8)

    h1, ff = _gather_l1(gidx, a_rows, p_rows, c_rows)
    pp = _mlp2(h1.reshape(ROWS * GROUP, 64), Wp2.T, bp2[None, :], Wp3.T, bp3[None, :])

    ff = ff.reshape(B, K, 64).transpose(0, 2, 1)
    pp = pp.reshape(B, K, 64).transpose(0, 2, 1)
    centroid_features = jnp.concatenate([ff, pp], axis=1)
    centroids = ctr_bk3.reshape(B, 3, K)
    return centroids, centroid_features


# speedup vs baseline: 12.1850x; 1.0936x over previous
import functools

import jax
import jax.numpy as jnp
import numpy as np
from jax import lax
from jax.experimental import pallas as pl
from jax.experimental.pallas import tpu as pltpu
from jax.experimental.pallas import tpu_sc as plsc

B = 8
N = 4096
CIN = 64
COUT = 128
K = N // 4
GROUP = 64
RAD2 = np.float32(0.2 ** 2)
NC, NS = 2, 16
NW = NC * NS
ROWS = B * K
RPW = ROWS // NW

_MAXI32 = np.int32(0x7FFFFFFF)
_RAD2_BITS = int(np.float32(0.2 ** 2).view(np.int32))


def _fps_body(pts_ref, cx_ref, cy_ref, cz_ref):
    X = pts_ref[:, 0, :]
    Y = pts_ref[:, 1, :]
    Z = pts_ref[:, 2, :]
    iota = lax.broadcasted_iota(jnp.int32, (B, N), 1)

    def extract(V, last):
        m = (iota == last).astype(jnp.float32)
        return jnp.sum(V * m, axis=1, keepdims=True)

    def step(t, carry):
        dists, last = carry
        cx = extract(X, last)
        cy = extract(Y, last)
        cz = extract(Z, last)
        cx_ref[pl.ds(t, 1), :] = cx.T
        cy_ref[pl.ds(t, 1), :] = cy.T
        cz_ref[pl.ds(t, 1), :] = cz.T
        dx = X - cx
        dy = Y - cy
        dz = Z - cz
        d = (dx * dx + dy * dy) + dz * dz
        dists = jnp.minimum(dists, d)
        mx = jnp.max(dists, axis=1, keepdims=True)
        nxt = jnp.min(jnp.where(dists == mx, iota, N), axis=1, keepdims=True)
        return dists, nxt

    init = (jnp.full((B, N), jnp.inf, jnp.float32), jnp.zeros((B, 1), jnp.int32))
    lax.fori_loop(0, K, step, init)


def _fps(points):
    out = pl.pallas_call(
        _fps_body,
        out_shape=[jax.ShapeDtypeStruct((K, B), jnp.float32)] * 3,
    )(points)
    return out


_KT = 256


def _d2_body(pts_ref, cx_ref, cy_ref, cz_ref, out_ref):
    X = pts_ref[0, 0, :][None, :]
    Y = pts_ref[0, 1, :][None, :]
    Z = pts_ref[0, 2, :][None, :]
    cx = cx_ref[0, 0, :][:, None]
    cy = cy_ref[0, 0, :][:, None]
    cz = cz_ref[0, 0, :][:, None]
    dx = cx - X
    dy = cy - Y
    dz = cz - Z
    out_ref[0] = (dx * dx + dy * dy) + dz * dz


def _d2(points, cx, cy, cz):
    grid = (B, K // _KT)
    cspec = pl.BlockSpec((1, 1, _KT), lambda b, k: (b, 0, k))
    return pl.pallas_call(
        _d2_body,
        grid=grid,
        in_specs=[
            pl.BlockSpec((1, 3, N), lambda b, k: (b, 0, 0)),
            cspec, cspec, cspec,
        ],
        out_specs=pl.BlockSpec((1, _KT, N), lambda b, k: (b, k, 0)),
        out_shape=jax.ShapeDtypeStruct((B, K, N), jnp.float32),
    )(points, cx, cy, cz)


def _sel_body(d_hbm, gidx_hbm, dbuf, vbuf, ibuf, sbuf, obuf, sem):
    wid = lax.axis_index("s") * NC + lax.axis_index("c")
    base = wid * RPW
    lane = lax.iota(jnp.int32, 16)
    zeros16 = jnp.zeros((16,), jnp.int32)

    one16 = jnp.full((16,), 1, jnp.int32)
    radv = jnp.full((16,), RAD2, jnp.float32)

    def row_body(g, _):
        row = base + g
        pltpu.async_copy(d_hbm.at[pl.ds(row * N, N)], dbuf, sem).wait()

        def cbody(j, cnt):
            v = dbuf[pl.ds(j * 16, 16)]
            m = v <= radv
            mi = jnp.where(m, one16, zeros16)
            inc = plsc.cumsum(mi)
            pos = cnt + inc - mi
            plsc.store_scatter(vbuf, [pos], v, mask=m)
            plsc.store_scatter(ibuf, [pos], lane + j * 16, mask=m)
            return cnt + plsc.all_reduce_population_count(m)

        cnt_vec = lax.fori_loop(0, N // 16, cbody, zeros16)
        cnt = cnt_vec[0]
        nv = (cnt + 15) >> 4

        def count_le(mid):
            def kbody(j, c):
                kv = plsc.bitcast(vbuf[pl.ds(j * 16, 16)], jnp.int32)
                valid = (lane + j * 16) < cnt_vec
                return c + plsc.all_reduce_population_count((kv <= mid) & valid)

            return lax.fori_loop(0, nv, kbody, zeros16)

        def bbody(_, lohi):
            lo, hi = lohi
            mid = (lo + hi) >> 1
            c = count_le(mid)
            upd = lo < hi
            ge = c >= 64
            hi = jnp.where(upd & ge, mid, hi)
            lo = jnp.where(upd & (~ge), mid + 1, lo)
            return lo, hi

        lo0 = zeros16
        hi0 = jnp.full((16,), _RAD2_BITS + 1, jnp.int32)
        t_vec, _hi = lax.fori_loop(0, 31, bbody, (lo0, hi0))
        t_vec = jnp.where(cnt_vec <= 64, jnp.full((16,), _MAXI32), t_vec)

        pad = ibuf[pl.ds(0, 16)][0]
        for q in range(4):
            obuf[pl.ds(q * 16, 16)] = jnp.full((16,), pad, jnp.int32)

        def emit_lt(j, c_lt):
            kv = plsc.bitcast(vbuf[pl.ds(j * 16, 16)], jnp.int32)
            iv = ibuf[pl.ds(j * 16, 16)]
            valid = (lane + j * 16) < cnt_vec
            m_lt = (kv < t_vec) & valid
            mi = jnp.where(m_lt, one16, zeros16)
            pos = c_lt + plsc.cumsum(mi) - mi
            plsc.store_scatter(obuf, [pos], iv, mask=m_lt)
            return c_lt + plsc.all_reduce_population_count(m_lt)

        c_lt = lax.fori_loop(0, nv, emit_lt, zeros16)

        cap64 = jnp.full((16,), 64, jnp.int32)

        def emit_eq(j, c_eq):
            kv = plsc.bitcast(vbuf[pl.ds(j * 16, 16)], jnp.int32)
            iv = ibuf[pl.ds(j * 16, 16)]
            valid = (lane + j * 16) < cnt_vec
            m_eq = (kv == t_vec) & valid
            me = jnp.where(m_eq, one16, zeros16)
            pos_e = c_eq + plsc.cumsum(me) - me
            plsc.store_scatter(obuf, [pos_e], iv, mask=m_eq & (pos_e < cap64))
            return c_eq + plsc.all_reduce_population_count(m_eq)

        lax.fori_loop(0, nv, emit_eq, c_lt)
        pltpu.async_copy(obuf, gidx_hbm.at[pl.ds(row * GROUP, GROUP)], sem).wait()
        return 0

    lax.fori_loop(0, RPW, row_body, 0)


def _select(d2):
    f = pl.kernel(
        _sel_body,
        out_type=jax.ShapeDtypeStruct((ROWS * GROUP,), jnp.int32),
        mesh=plsc.VectorSubcoreMesh(core_axis_name="c", subcore_axis_name="s"),
        compiler_params=pltpu.CompilerParams(needs_layout_passes=False),
        scratch_types=[
            pltpu.VMEM((N,), jnp.float32),
            pltpu.VMEM((N,), jnp.float32),
            pltpu.VMEM((N,), jnp.int32),
            pltpu.VMEM((16,), jnp.int32),
            pltpu.VMEM((GROUP,), jnp.int32),
            pltpu.SemaphoreType.DMA,
        ],
    )
    return f(d2.reshape(ROWS * N))


_RT = 2048


def _pre_body(ft_ref, pt_ref, wf1, bf1, wf2, bf2, wf3, bf3, wp1, bp1, p_ref, a_ref):
    h = jnp.maximum(jnp.dot(ft_ref[...], wf1[...], preferred_element_type=jnp.float32) + bf1[...], 0.0)
    h = jnp.maximum(jnp.dot(h, wf2[...], preferred_element_type=jnp.float32) + bf2[...], 0.0)
    p_ref[...] = jnp.dot(h, wf3[...], preferred_element_type=jnp.float32) + bf3[...]
    a_ref[...] = jnp.dot(pt_ref[...], wp1[...], preferred_element_type=jnp.float32) + bp1[...]


def _precompute(ft, pt8, wf1t, bf1, wf2t, bf2, wf3t, bf3, wp1t8, bp1):
    grid = (B * N // _RT,)
    wspec = pl.BlockSpec((64, 64), lambda i: (0, 0))
    bspec = pl.BlockSpec((1, 64), lambda i: (0, 0))
    return pl.pallas_call(
        _pre_body,
        grid=grid,
        in_specs=[
            pl.BlockSpec((_RT, 64), lambda i: (i, 0)),
            pl.BlockSpec((_RT, 8), lambda i: (i, 0)),
            wspec, bspec, wspec, bspec, wspec, bspec,
            pl.BlockSpec((8, 64), lambda i: (0, 0)), bspec,
        ],
        out_specs=[
            pl.BlockSpec((_RT, 64), lambda i: (i, 0)),
            pl.BlockSpec((_RT, 64), lambda i: (i, 0)),
        ],
        out_shape=[
            jax.ShapeDtypeStruct((B * N, 64), jnp.float32),
            jax.ShapeDtypeStruct((B * N, 64), jnp.float32),
        ],
    )(ft, pt8, wf1t, bf1, wf2t, bf2, wf3t, bf3, wp1t8, bp1)


def _cmat_body(ctr_ref, wp1_ref, c_ref):
    c_ref[...] = jnp.dot(ctr_ref[...], wp1_ref[...], preferred_element_type=jnp.float32)


def _cmat(ctr8, wp1t8):
    return pl.pallas_call(
        _cmat_body,
        out_shape=jax.ShapeDtypeStruct((ROWS, 64), jnp.float32),
    )(ctr8, wp1t8)


def _gath_body(gidx_hbm, a_hbm, p_hbm, c_hbm, h1_hbm, ff_hbm,
               idx0, idx1, ab0, ab1, pb0, pb1, cb0, cb1, h10, h11, fb0, fb1,
               si0, si1, sa0, sa1, sp0, sp1, sc0, sc1, sw0, sw1, sw2, sw3):
    wid = lax.axis_index("s") * NC + lax.axis_index("c")
    base = wid * RPW
    neg_inf = jnp.full((16,), -jnp.inf, jnp.float32)
    bufs = ((idx0, ab0, pb0, cb0, h10, fb0, sa0, sp0, sc0, sw0, sw1),
            (idx1, ab1, pb1, cb1, h11, fb1, sa1, sp1, sc1, sw2, sw3))

    def pair_body(it, _):
        g = it * 2
        cpi = []
        for par in range(2):
            row = base + g + par
            cpi.append(pltpu.async_copy(
                gidx_hbm.at[pl.ds(row * GROUP, GROUP)], bufs[par][0],
                (si0, si1)[par]))
        gathers = []
        for par in range(2):
            row = base + g + par
            idxb, abuf, pbuf, cbuf = bufs[par][:4]
            sa, sp, sc = bufs[par][6:9]
            cpi[par].wait()
            boffv = jnp.full((16,), (row >> 10) * N, jnp.int32)
            for q in range(4):
                idxb[pl.ds(q * 16, 16)] = idxb[pl.ds(q * 16, 16)] + boffv
            gathers.append((
                pltpu.async_copy(a_hbm.at[idxb], abuf, sa),
                pltpu.async_copy(p_hbm.at[idxb], pbuf, sp),
                pltpu.async_copy(c_hbm.at[pl.ds(row * 64, 64)], cbuf, sc)))
        writes = []
        for par in range(2):
            row = base + g + par
            idxb, abuf, pbuf, cbuf, h1buf, ffbuf = bufs[par][:6]
            swh, swf = bufs[par][9:11]
            for cp in gathers[par]:
                cp.wait()

            def rbody(r, acc):
                out = []
                for q in range(4):
                    a = abuf[r, pl.ds(q * 16, 16)]
                    c = cbuf[pl.ds(q * 16, 16)]
                    h1buf[pl.ds(r * 64 + q * 16, 16)] = jnp.maximum(a - c, 0.0)
                    out.append(jnp.maximum(acc[q], pbuf[r, pl.ds(q * 16, 16)]))
                return tuple(out)

            acc = lax.fori_loop(0, GROUP, rbody, (neg_inf,) * 4)
            for q in range(4):
                ffbuf[pl.ds(q * 16, 16)] = acc[q]
            writes.append(pltpu.async_copy(
                h1buf, h1_hbm.at[pl.ds(row * GROUP * 64, GROUP * 64)], swh))
            writes.append(pltpu.async_copy(
                ffbuf, ff_hbm.at[pl.ds(row * 64, 64)], swf))
        for cp in writes:
            cp.wait()
        return 0

    lax.fori_loop(0, RPW // 2, pair_body, 0)


def _gather_l1(gidx, a_rows, p_rows, c_rows):
    f = pl.kernel(
        _gath_body,
        out_type=[
            jax.ShapeDtypeStruct((ROWS * GROUP * 64,), jnp.float32),
            jax.ShapeDtypeStruct((ROWS * 64,), jnp.float32),
        ],
        mesh=plsc.VectorSubcoreMesh(core_axis_name="c", subcore_axis_name="s"),
        compiler_params=pltpu.CompilerParams(needs_layout_passes=False,
                                             use_tc_tiling_on_sc=False),
        scratch_types=(
            [pltpu.VMEM((GROUP,), jnp.int32)] * 2
            + [pltpu.VMEM((GROUP, 64), jnp.float32)] * 4
            + [pltpu.VMEM((64,), jnp.float32)] * 2
            + [pltpu.VMEM((GROUP * 64,), jnp.float32)] * 2
            + [pltpu.VMEM((64,), jnp.float32)] * 2
            + [pltpu.SemaphoreType.DMA] * 12
        ),
    )
    return f(gidx, a_rows, p_rows, c_rows.reshape(ROWS * 64))


_GT = 64


def _mlp2_body(h1_ref, wp2, bp2, wp3, bp3, out_ref):
    h = jnp.maximum(jnp.dot(h1_ref[...], wp2[...], preferred_element_type=jnp.float32) + bp2[...], 0.0)
    h = jnp.dot(h, wp3[...], preferred_element_type=jnp.float32) + bp3[...]
    out_ref[...] = jnp.max(h.reshape(_GT, GROUP, 64), axis=1)


def _mlp2(h1, wp2t, bp2, wp3t, bp3):
    grid = (ROWS // _GT,)
    wspec = pl.BlockSpec((64, 64), lambda i: (0, 0))
    bspec = pl.BlockSpec((1, 64), lambda i: (0, 0))
    return pl.pallas_call(
        _mlp2_body,
        grid=grid,
        in_specs=[
            pl.BlockSpec((_GT * GROUP, 64), lambda i: (i, 0)),
            wspec, bspec, wspec, bspec,
        ],
        out_specs=pl.BlockSpec((_GT, 64), lambda i: (i, 0)),
        out_shape=jax.ShapeDtypeStruct((ROWS, 64), jnp.float32),
    )(h1, wp2t, bp2, wp3t, bp3)


def kernel(points, point_features, Wp1, bp1, Wp2, bp2, Wp3, bp3,
           Wf1, bf1, Wf2, bf2, Wf3, bf3):
    ctrx, ctry, ctrz = _fps(points)
    cx, cy, cz = ctrx.T, ctry.T, ctrz.T
    d2 = _d2(points, cx[:, None, :], cy[:, None, :], cz[:, None, :])
    gidx = _select(d2)

    ft = point_features.transpose(0, 2, 1).reshape(B * N, CIN)
    pt = points.transpose(0, 2, 1).reshape(B * N, 3)
    pt8 = jnp.pad(pt, ((0, 0), (0, 5)))
    wp1t8 = jnp.pad(Wp1.T, ((0, 5), (0, 0)))
    p_rows, a_rows = _precompute(
        ft, pt8, Wf1.T, bf1[None, :], Wf2.T, bf2[None, :], Wf3.T, bf3[None, :],
        wp1t8, bp1[None, :])

    ctr_bk3 = jnp.stack([cx, cy, cz], axis=-1)
    ctr8 = jnp.pad(ctr_bk3.reshape(ROWS, 3), ((0, 0), (0, 5)))
    c_rows = _cmat(ctr8, wp1t8)

    h1, ff = _gather_l1(gidx, a_rows, p_rows, c_rows)
    pp = _mlp2(h1.reshape(ROWS * GROUP, 64), Wp2.T, bp2[None, :], Wp3.T, bp3[None, :])

    ff = ff.reshape(B, K, 64).transpose(0, 2, 1)
    pp = pp.reshape(B, K, 64).transpose(0, 2, 1)
    centroid_features = jnp.concatenate([ff, pp], axis=1)
    centroids = ctr_bk3.reshape(B, 3, K)
    return centroids, centroid_features


# speedup vs baseline: 12.5442x; 1.0295x over previous
import functools

import jax
import jax.numpy as jnp
import numpy as np
from jax import lax
from jax.experimental import pallas as pl
from jax.experimental.pallas import tpu as pltpu
from jax.experimental.pallas import tpu_sc as plsc

B = 8
N = 4096
CIN = 64
COUT = 128
K = N // 4
GROUP = 64
RAD2 = np.float32(0.2 ** 2)
NC, NS = 2, 16
NW = NC * NS
ROWS = B * K
RPW = ROWS // NW

_MAXI32 = np.int32(0x7FFFFFFF)
_RAD2_BITS = int(np.float32(0.2 ** 2).view(np.int32))


def _fps_body(pts_ref, cx_ref, cy_ref, cz_ref):
    X = pts_ref[:, 0, :]
    Y = pts_ref[:, 1, :]
    Z = pts_ref[:, 2, :]
    iota = lax.broadcasted_iota(jnp.int32, (B, N), 1)

    def extract(V, last):
        m = (iota == last).astype(jnp.float32)
        return jnp.sum(V * m, axis=1, keepdims=True)

    def step(t, carry):
        dists, last = carry
        cx = extract(X, last)
        cy = extract(Y, last)
        cz = extract(Z, last)
        cx_ref[pl.ds(t, 1), :] = cx.T
        cy_ref[pl.ds(t, 1), :] = cy.T
        cz_ref[pl.ds(t, 1), :] = cz.T
        dx = X - cx
        dy = Y - cy
        dz = Z - cz
        d = (dx * dx + dy * dy) + dz * dz
        dists = jnp.minimum(dists, d)
        mx = jnp.max(dists, axis=1, keepdims=True)
        nxt = jnp.min(jnp.where(dists == mx, iota, N), axis=1, keepdims=True)
        return dists, nxt

    init = (jnp.full((B, N), jnp.inf, jnp.float32), jnp.zeros((B, 1), jnp.int32))
    lax.fori_loop(0, K, step, init)


def _fps(points):
    out = pl.pallas_call(
        _fps_body,
        out_shape=[jax.ShapeDtypeStruct((K, B), jnp.float32)] * 3,
    )(points)
    return out


_KT = 256


def _d2_body(pts_ref, cx_ref, cy_ref, cz_ref, out_ref):
    X = pts_ref[0, 0, :][None, :]
    Y = pts_ref[0, 1, :][None, :]
    Z = pts_ref[0, 2, :][None, :]
    cx = cx_ref[0, 0, :][:, None]
    cy = cy_ref[0, 0, :][:, None]
    cz = cz_ref[0, 0, :][:, None]
    dx = cx - X
    dy = cy - Y
    dz = cz - Z
    out_ref[0] = (dx * dx + dy * dy) + dz * dz


def _d2(points, cx, cy, cz):
    grid = (B, K // _KT)
    cspec = pl.BlockSpec((1, 1, _KT), lambda b, k: (b, 0, k))
    return pl.pallas_call(
        _d2_body,
        grid=grid,
        in_specs=[
            pl.BlockSpec((1, 3, N), lambda b, k: (b, 0, 0)),
            cspec, cspec, cspec,
        ],
        out_specs=pl.BlockSpec((1, _KT, N), lambda b, k: (b, k, 0)),
        out_shape=jax.ShapeDtypeStruct((B, K, N), jnp.float32),
    )(points, cx, cy, cz)


def _sel_body(d_hbm, gidx_hbm, dbuf, vbuf, ibuf, sbuf, obuf, sem):
    wid = lax.axis_index("s") * NC + lax.axis_index("c")
    base = wid * RPW
    lane = lax.iota(jnp.int32, 16)
    zeros16 = jnp.zeros((16,), jnp.int32)

    one16 = jnp.full((16,), 1, jnp.int32)
    radv = jnp.full((16,), RAD2, jnp.float32)

    def row_body(g, _):
        row = base + g
        pltpu.async_copy(d_hbm.at[pl.ds(row * N, N)], dbuf, sem).wait()

        def cbody(j, cnt):
            for u in range(2):
                v = dbuf[pl.ds(j * 32 + u * 16, 16)]
                m = v <= radv
                mi = jnp.where(m, one16, zeros16)
                inc = plsc.cumsum(mi)
                pos = cnt + inc - mi
                plsc.store_scatter(vbuf, [pos], v, mask=m)
                plsc.store_scatter(ibuf, [pos], lane + (j * 32 + u * 16), mask=m)
                cnt = cnt + plsc.all_reduce_population_count(m)
            return cnt

        cnt_vec = lax.fori_loop(0, N // 32, cbody, zeros16)
        cnt = cnt_vec[0]
        nv = (cnt + 15) >> 4

        nv2 = (cnt + 31) >> 5

        def count_le(mid):
            def kbody(j, c):
                for u in range(2):
                    kv = plsc.bitcast(vbuf[pl.ds(j * 32 + u * 16, 16)], jnp.int32)
                    valid = (lane + (j * 32 + u * 16)) < cnt_vec
                    c = c + plsc.all_reduce_population_count((kv <= mid) & valid)
                return c

            return lax.fori_loop(0, nv2, kbody, zeros16)

        def bbody(_, lohi):
            lo, hi = lohi
            mid = (lo + hi) >> 1
            c = count_le(mid)
            upd = lo < hi
            ge = c >= 64
            hi = jnp.where(upd & ge, mid, hi)
            lo = jnp.where(upd & (~ge), mid + 1, lo)
            return lo, hi

        lo0 = zeros16
        hi0 = jnp.full((16,), _RAD2_BITS + 1, jnp.int32)
        nsteps = jnp.where(cnt > 64, 31, 0)
        t_vec, _hi = lax.fori_loop(0, nsteps, bbody, (lo0, hi0))
        t_vec = jnp.where(cnt_vec <= 64, jnp.full((16,), _MAXI32), t_vec)

        pad = ibuf[pl.ds(0, 16)][0]
        for q in range(4):
            obuf[pl.ds(q * 16, 16)] = jnp.full((16,), pad, jnp.int32)

        def emit_lt(j, c_lt):
            kv = plsc.bitcast(vbuf[pl.ds(j * 16, 16)], jnp.int32)
            iv = ibuf[pl.ds(j * 16, 16)]
            valid = (lane + j * 16) < cnt_vec
            m_lt = (kv < t_vec) & valid
            mi = jnp.where(m_lt, one16, zeros16)
            pos = c_lt + plsc.cumsum(mi) - mi
            plsc.store_scatter(obuf, [pos], iv, mask=m_lt)
            return c_lt + plsc.all_reduce_population_count(m_lt)

        c_lt = lax.fori_loop(0, nv, emit_lt, zeros16)

        cap64 = jnp.full((16,), 64, jnp.int32)

        def emit_eq(j, c_eq):
            kv = plsc.bitcast(vbuf[pl.ds(j * 16, 16)], jnp.int32)
            iv = ibuf[pl.ds(j * 16, 16)]
            valid = (lane + j * 16) < cnt_vec
            m_eq = (kv == t_vec) & valid
            me = jnp.where(m_eq, one16, zeros16)
            pos_e = c_eq + plsc.cumsum(me) - me
            plsc.store_scatter(obuf, [pos_e], iv, mask=m_eq & (pos_e < cap64))
            return c_eq + plsc.all_reduce_population_count(m_eq)

        lax.fori_loop(0, nv, emit_eq, c_lt)
        pltpu.async_copy(obuf, gidx_hbm.at[pl.ds(row * GROUP, GROUP)], sem).wait()
        return 0

    lax.fori_loop(0, RPW, row_body, 0)


def _select(d2):
    f = pl.kernel(
        _sel_body,
        out_type=jax.ShapeDtypeStruct((ROWS * GROUP,), jnp.int32),
        mesh=plsc.VectorSubcoreMesh(core_axis_name="c", subcore_axis_name="s"),
        compiler_params=pltpu.CompilerParams(needs_layout_passes=False),
        scratch_types=[
            pltpu.VMEM((N,), jnp.float32),
            pltpu.VMEM((N,), jnp.float32),
            pltpu.VMEM((N,), jnp.int32),
            pltpu.VMEM((16,), jnp.int32),
            pltpu.VMEM((GROUP,), jnp.int32),
            pltpu.SemaphoreType.DMA,
        ],
    )
    return f(d2.reshape(ROWS * N))


_RT = 2048


def _pre_body(ft_ref, pt_ref, wf1, bf1, wf2, bf2, wf3, bf3, wp1, bp1, p_ref, a_ref):
    h = jnp.maximum(jnp.dot(ft_ref[...], wf1[...], preferred_element_type=jnp.float32) + bf1[...], 0.0)
    h = jnp.maximum(jnp.dot(h, wf2[...], preferred_element_type=jnp.float32) + bf2[...], 0.0)
    p_ref[...] = jnp.dot(h, wf3[...], preferred_element_type=jnp.float32) + bf3[...]
    a_ref[...] = jnp.dot(pt_ref[...], wp1[...], preferred_element_type=jnp.float32) + bp1[...]


def _precompute(ft, pt8, wf1t, bf1, wf2t, bf2, wf3t, bf3, wp1t8, bp1):
    grid = (B * N // _RT,)
    wspec = pl.BlockSpec((64, 64), lambda i: (0, 0))
    bspec = pl.BlockSpec((1, 64), lambda i: (0, 0))
    return pl.pallas_call(
        _pre_body,
        grid=grid,
        in_specs=[
            pl.BlockSpec((_RT, 64), lambda i: (i, 0)),
            pl.BlockSpec((_RT, 8), lambda i: (i, 0)),
            wspec, bspec, wspec, bspec, wspec, bspec,
            pl.BlockSpec((8, 64), lambda i: (0, 0)), bspec,
        ],
        out_specs=[
            pl.BlockSpec((_RT, 64), lambda i: (i, 0)),
            pl.BlockSpec((_RT, 64), lambda i: (i, 0)),
        ],
        out_shape=[
            jax.ShapeDtypeStruct((B * N, 64), jnp.float32),
            jax.ShapeDtypeStruct((B * N, 64), jnp.float32),
        ],
    )(ft, pt8, wf1t, bf1, wf2t, bf2, wf3t, bf3, wp1t8, bp1)


def _cmat_body(ctr_ref, wp1_ref, c_ref):
    c_ref[...] = jnp.dot(ctr_ref[...], wp1_ref[...], preferred_element_type=jnp.float32)


def _cmat(ctr8, wp1t8):
    return pl.pallas_call(
        _cmat_body,
        out_shape=jax.ShapeDtypeStruct((ROWS, 64), jnp.float32),
    )(ctr8, wp1t8)


def _gath_body(gidx_hbm, a_hbm, p_hbm, c_hbm, h1_hbm, ff_hbm,
               idx0, idx1, ab0, ab1, pb0, pb1, cb0, cb1, h10, h11, fb0, fb1,
               si0, si1, sa0, sa1, sp0, sp1, sc0, sc1, sw0, sw1, sw2, sw3):
    wid = lax.axis_index("s") * NC + lax.axis_index("c")
    base = wid * RPW
    neg_inf = jnp.full((16,), -jnp.inf, jnp.float32)
    bufs = ((idx0, ab0, pb0, cb0, h10, fb0, sa0, sp0, sc0, sw0, sw1),
            (idx1, ab1, pb1, cb1, h11, fb1, sa1, sp1, sc1, sw2, sw3))

    def pair_body(it, _):
        g = it * 2
        cpi = []
        for par in range(2):
            row = base + g + par
            cpi.append(pltpu.async_copy(
                gidx_hbm.at[pl.ds(row * GROUP, GROUP)], bufs[par][0],
                (si0, si1)[par]))
        gathers = []
        for par in range(2):
            row = base + g + par
            idxb, abuf, pbuf, cbuf = bufs[par][:4]
            sa, sp, sc = bufs[par][6:9]
            cpi[par].wait()
            boffv = jnp.full((16,), (row >> 10) * N, jnp.int32)
            for q in range(4):
                idxb[pl.ds(q * 16, 16)] = idxb[pl.ds(q * 16, 16)] + boffv
            gathers.append((
                pltpu.async_copy(a_hbm.at[idxb], abuf, sa),
                pltpu.async_copy(p_hbm.at[idxb], pbuf, sp),
                pltpu.async_copy(c_hbm.at[pl.ds(row * 64, 64)], cbuf, sc)))
        writes = []
        for par in range(2):
            row = base + g + par
            idxb, abuf, pbuf, cbuf, h1buf, ffbuf = bufs[par][:6]
            swh, swf = bufs[par][9:11]
            for cp in gathers[par]:
                cp.wait()

            def rbody(r, acc):
                out = []
                for q in range(4):
                    a = abuf[r, pl.ds(q * 16, 16)]
                    c = cbuf[pl.ds(q * 16, 16)]
                    h1buf[pl.ds(r * 64 + q * 16, 16)] = jnp.maximum(a - c, 0.0)
                    out.append(jnp.maximum(acc[q], pbuf[r, pl.ds(q * 16, 16)]))
                return tuple(out)

            acc = lax.fori_loop(0, GROUP, rbody, (neg_inf,) * 4)
            for q in range(4):
                ffbuf[pl.ds(q * 16, 16)] = acc[q]
            writes.append(pltpu.async_copy(
                h1buf, h1_hbm.at[pl.ds(row * GROUP * 64, GROUP * 64)], swh))
            writes.append(pltpu.async_copy(
                ffbuf, ff_hbm.at[pl.ds(row * 64, 64)], swf))
        for cp in writes:
            cp.wait()
        return 0

    lax.fori_loop(0, RPW // 2, pair_body, 0)


def _gather_l1(gidx, a_rows, p_rows, c_rows):
    f = pl.kernel(
        _gath_body,
        out_type=[
            jax.ShapeDtypeStruct((ROWS * GROUP * 64,), jnp.float32),
            jax.ShapeDtypeStruct((ROWS * 64,), jnp.float32),
        ],
        mesh=plsc.VectorSubcoreMesh(core_axis_name="c", subcore_axis_name="s"),
        compiler_params=pltpu.CompilerParams(needs_layout_passes=False,
                                             use_tc_tiling_on_sc=False),
        scratch_types=(
            [pltpu.VMEM((GROUP,), jnp.int32)] * 2
            + [pltpu.VMEM((GROUP, 64), jnp.float32)] * 4
            + [pltpu.VMEM((64,), jnp.float32)] * 2
            + [pltpu.VMEM((GROUP * 64,), jnp.float32)] * 2
            + [pltpu.VMEM((64,), jnp.float32)] * 2
            + [pltpu.SemaphoreType.DMA] * 12
        ),
    )
    return f(gidx, a_rows, p_rows, c_rows.reshape(ROWS * 64))


_GT = 64


def _mlp2_body(h1_ref, wp2, bp2, wp3, bp3, out_ref):
    h = jnp.maximum(jnp.dot(h1_ref[...], wp2[...], preferred_element_type=jnp.float32) + bp2[...], 0.0)
    h = jnp.dot(h, wp3[...], preferred_element_type=jnp.float32) + bp3[...]
    out_ref[...] = jnp.max(h.reshape(_GT, GROUP, 64), axis=1)


def _mlp2(h1, wp2t, bp2, wp3t, bp3):
    grid = (ROWS // _GT,)
    wspec = pl.BlockSpec((64, 64), lambda i: (0, 0))
    bspec = pl.BlockSpec((1, 64), lambda i: (0, 0))
    return pl.pallas_call(
        _mlp2_body,
        grid=grid,
        in_specs=[
            pl.BlockSpec((_GT * GROUP, 64), lambda i: (i, 0)),
            wspec, bspec, wspec, bspec,
        ],
        out_specs=pl.BlockSpec((_GT, 64), lambda i: (i, 0)),
        out_shape=jax.ShapeDtypeStruct((ROWS, 64), jnp.float32),
    )(h1, wp2t, bp2, wp3t, bp3)


def kernel(points, point_features, Wp1, bp1, Wp2, bp2, Wp3, bp3,
           Wf1, bf1, Wf2, bf2, Wf3, bf3):
    ctrx, ctry, ctrz = _fps(points)
    cx, cy, cz = ctrx.T, ctry.T, ctrz.T
    d2 = _d2(points, cx[:, None, :], cy[:, None, :], cz[:, None, :])
    gidx = _select(d2)

    ft = point_features.transpose(0, 2, 1).reshape(B * N, CIN)
    pt = points.transpose(0, 2, 1).reshape(B * N, 3)
    pt8 = jnp.pad(pt, ((0, 0), (0, 5)))
    wp1t8 = jnp.pad(Wp1.T, ((0, 5), (0, 0)))
    p_rows, a_rows = _precompute(
        ft, pt8, Wf1.T, bf1[None, :], Wf2.T, bf2[None, :], Wf3.T, bf3[None, :],
        wp1t8, bp1[None, :])

    ctr_bk3 = jnp.stack([cx, cy, cz], axis=-1)
    ctr8 = jnp.pad(ctr_bk3.reshape(ROWS, 3), ((0, 0), (0, 5)))
    c_rows = _cmat(ctr8, wp1t8)

    h1, ff = _gather_l1(gidx, a_rows, p_rows, c_rows)
    pp = _mlp2(h1.reshape(ROWS * GROUP, 64), Wp2.T, bp2[None, :], Wp3.T, bp3[None, :])

    ff = ff.reshape(B, K, 64).transpose(0, 2, 1)
    pp = pp.reshape(B, K, 64).transpose(0, 2, 1)
    centroid_features = jnp.concatenate([ff, pp], axis=1)
    centroids = ctr_bk3.reshape(B, 3, K)
    return centroids, centroid_features


# speedup vs baseline: 12.9451x; 1.0320x over previous
import functools

import jax
import jax.numpy as jnp
import numpy as np
from jax import lax
from jax.experimental import pallas as pl
from jax.experimental.pallas import tpu as pltpu
from jax.experimental.pallas import tpu_sc as plsc

B = 8
N = 4096
CIN = 64
COUT = 128
K = N // 4
GROUP = 64
RAD2 = np.float32(0.2 ** 2)
NC, NS = 2, 16
NW = NC * NS
ROWS = B * K
RPW = ROWS // NW

_MAXI32 = np.int32(0x7FFFFFFF)
_RAD2_BITS = int(np.float32(0.2 ** 2).view(np.int32))


def _fps_body(pts_ref, cx_ref, cy_ref, cz_ref):
    X = pts_ref[:, 0, :]
    Y = pts_ref[:, 1, :]
    Z = pts_ref[:, 2, :]
    iota = lax.broadcasted_iota(jnp.int32, (B, N), 1)

    def extract(V, last):
        m = (iota == last).astype(jnp.float32)
        return jnp.sum(V * m, axis=1, keepdims=True)

    def step(t, carry):
        dists, last = carry
        cx = extract(X, last)
        cy = extract(Y, last)
        cz = extract(Z, last)
        cx_ref[pl.ds(t, 1), :] = cx.T
        cy_ref[pl.ds(t, 1), :] = cy.T
        cz_ref[pl.ds(t, 1), :] = cz.T
        dx = X - cx
        dy = Y - cy
        dz = Z - cz
        d = (dx * dx + dy * dy) + dz * dz
        dists = jnp.minimum(dists, d)
        mx = jnp.max(dists, axis=1, keepdims=True)
        nxt = jnp.min(jnp.where(dists == mx, iota, N), axis=1, keepdims=True)
        return dists, nxt

    init = (jnp.full((B, N), jnp.inf, jnp.float32), jnp.zeros((B, 1), jnp.int32))
    lax.fori_loop(0, K, step, init)


def _fps(points):
    out = pl.pallas_call(
        _fps_body,
        out_shape=[jax.ShapeDtypeStruct((K, B), jnp.float32)] * 3,
    )(points)
    return out


_KT = 256


def _d2_body(pts_ref, cx_ref, cy_ref, cz_ref, out_ref):
    X = pts_ref[0, 0, :][None, :]
    Y = pts_ref[0, 1, :][None, :]
    Z = pts_ref[0, 2, :][None, :]
    cx = cx_ref[0, 0, :][:, None]
    cy = cy_ref[0, 0, :][:, None]
    cz = cz_ref[0, 0, :][:, None]
    dx = cx - X
    dy = cy - Y
    dz = cz - Z
    out_ref[0] = (dx * dx + dy * dy) + dz * dz


def _d2(points, cx, cy, cz):
    grid = (B, K // _KT)
    cspec = pl.BlockSpec((1, 1, _KT), lambda b, k: (b, 0, k))
    return pl.pallas_call(
        _d2_body,
        grid=grid,
        in_specs=[
            pl.BlockSpec((1, 3, N), lambda b, k: (b, 0, 0)),
            cspec, cspec, cspec,
        ],
        out_specs=pl.BlockSpec((1, _KT, N), lambda b, k: (b, k, 0)),
        out_shape=jax.ShapeDtypeStruct((B, K, N), jnp.float32),
    )(points, cx, cy, cz)


def _sel_body(d_hbm, gidx_hbm, dbuf, dbuf1, vbuf, ibuf, sbuf, obuf, sem, sem1, semo):
    wid = lax.axis_index("s") * NC + lax.axis_index("c")
    base = wid * RPW
    lane = lax.iota(jnp.int32, 16)
    zeros16 = jnp.zeros((16,), jnp.int32)

    one16 = jnp.full((16,), 1, jnp.int32)
    radv = jnp.full((16,), RAD2, jnp.float32)

    def pair_body(it, _):
        g = it * 2
        cp0 = pltpu.async_copy(d_hbm.at[pl.ds((base + g) * N, N)], dbuf, sem)
        cp1 = pltpu.async_copy(d_hbm.at[pl.ds((base + g + 1) * N, N)], dbuf1, sem1)
        cp0.wait()
        row_body(base + g, dbuf, cp1)
        return 0

    def row_body(row, dbuf, cp_next):

        def cbody(j, cnt):
            for u in range(2):
                v = dbuf[pl.ds(j * 32 + u * 16, 16)]
                m = v <= radv
                mi = jnp.where(m, one16, zeros16)
                inc = plsc.cumsum(mi)
                pos = cnt + inc - mi
                plsc.store_scatter(vbuf, [pos], v, mask=m)
                plsc.store_scatter(ibuf, [pos], lane + (j * 32 + u * 16), mask=m)
                cnt = cnt + plsc.all_reduce_population_count(m)
            return cnt

        cnt_vec = lax.fori_loop(0, N // 32, cbody, zeros16)
        cnt = cnt_vec[0]
        nv = (cnt + 15) >> 4

        nv2 = (cnt + 31) >> 5

        def count_le(mid):
            def kbody(j, c):
                for u in range(2):
                    kv = plsc.bitcast(vbuf[pl.ds(j * 32 + u * 16, 16)], jnp.int32)
                    valid = (lane + (j * 32 + u * 16)) < cnt_vec
                    c = c + plsc.all_reduce_population_count((kv <= mid) & valid)
                return c

            return lax.fori_loop(0, nv2, kbody, zeros16)

        def bbody(_, lohi):
            lo, hi = lohi
            mid = (lo + hi) >> 1
            c = count_le(mid)
            upd = lo < hi
            ge = c >= 64
            hi = jnp.where(upd & ge, mid, hi)
            lo = jnp.where(upd & (~ge), mid + 1, lo)
            return lo, hi

        lo0 = zeros16
        hi0 = jnp.full((16,), _RAD2_BITS + 1, jnp.int32)
        nsteps = jnp.where(cnt > 64, 31, 0)
        t_vec, _hi = lax.fori_loop(0, nsteps, bbody, (lo0, hi0))
        t_vec = jnp.where(cnt_vec <= 64, jnp.full((16,), _MAXI32), t_vec)

        pad = ibuf[pl.ds(0, 16)][0]
        for q in range(4):
            obuf[pl.ds(q * 16, 16)] = jnp.full((16,), pad, jnp.int32)

        def emit_lt(j, c_lt):
            kv = plsc.bitcast(vbuf[pl.ds(j * 16, 16)], jnp.int32)
            iv = ibuf[pl.ds(j * 16, 16)]
            valid = (lane + j * 16) < cnt_vec
            m_lt = (kv < t_vec) & valid
            mi = jnp.where(m_lt, one16, zeros16)
            pos = c_lt + plsc.cumsum(mi) - mi
            plsc.store_scatter(obuf, [pos], iv, mask=m_lt)
            return c_lt + plsc.all_reduce_population_count(m_lt)

        c_lt = lax.fori_loop(0, nv, emit_lt, zeros16)

        cap64 = jnp.full((16,), 64, jnp.int32)

        def emit_eq(j, c_eq):
            kv = plsc.bitcast(vbuf[pl.ds(j * 16, 16)], jnp.int32)
            iv = ibuf[pl.ds(j * 16, 16)]
            valid = (lane + j * 16) < cnt_vec
            m_eq = (kv == t_vec) & valid
            me = jnp.where(m_eq, one16, zeros16)
            pos_e = c_eq + plsc.cumsum(me) - me
            plsc.store_scatter(obuf, [pos_e], iv, mask=m_eq & (pos_e < cap64))
            return c_eq + plsc.all_reduce_population_count(m_eq)

        lax.fori_loop(0, nv, emit_eq, c_lt)
        pltpu.async_copy(obuf, gidx_hbm.at[pl.ds(row * GROUP, GROUP)], semo).wait()
        if cp_next is not None:
            cp_next.wait()
            row_body(row + 1, dbuf1, None)

    lax.fori_loop(0, RPW // 2, pair_body, 0)


def _select(d2):
    f = pl.kernel(
        _sel_body,
        out_type=jax.ShapeDtypeStruct((ROWS * GROUP,), jnp.int32),
        mesh=plsc.VectorSubcoreMesh(core_axis_name="c", subcore_axis_name="s"),
        compiler_params=pltpu.CompilerParams(needs_layout_passes=False),
        scratch_types=[
            pltpu.VMEM((N,), jnp.float32),
            pltpu.VMEM((N,), jnp.float32),
            pltpu.VMEM((N,), jnp.float32),
            pltpu.VMEM((N,), jnp.int32),
            pltpu.VMEM((16,), jnp.int32),
            pltpu.VMEM((GROUP,), jnp.int32),
            pltpu.SemaphoreType.DMA,
            pltpu.SemaphoreType.DMA,
            pltpu.SemaphoreType.DMA,
        ],
    )
    return f(d2.reshape(ROWS * N))


_RT = 2048


def _pre_body(ft_ref, pt_ref, wf1, bf1, wf2, bf2, wf3, bf3, wp1, bp1, p_ref, a_ref):
    h = jnp.maximum(jnp.dot(ft_ref[...], wf1[...], preferred_element_type=jnp.float32) + bf1[...], 0.0)
    h = jnp.maximum(jnp.dot(h, wf2[...], preferred_element_type=jnp.float32) + bf2[...], 0.0)
    p_ref[...] = jnp.dot(h, wf3[...], preferred_element_type=jnp.float32) + bf3[...]
    a_ref[...] = jnp.dot(pt_ref[...], wp1[...], preferred_element_type=jnp.float32) + bp1[...]


def _precompute(ft, pt8, wf1t, bf1, wf2t, bf2, wf3t, bf3, wp1t8, bp1):
    grid = (B * N // _RT,)
    wspec = pl.BlockSpec((64, 64), lambda i: (0, 0))
    bspec = pl.BlockSpec((1, 64), lambda i: (0, 0))
    return pl.pallas_call(
        _pre_body,
        grid=grid,
        in_specs=[
            pl.BlockSpec((_RT, 64), lambda i: (i, 0)),
            pl.BlockSpec((_RT, 8), lambda i: (i, 0)),
            wspec, bspec, wspec, bspec, wspec, bspec,
            pl.BlockSpec((8, 64), lambda i: (0, 0)), bspec,
        ],
        out_specs=[
            pl.BlockSpec((_RT, 64), lambda i: (i, 0)),
            pl.BlockSpec((_RT, 64), lambda i: (i, 0)),
        ],
        out_shape=[
            jax.ShapeDtypeStruct((B * N, 64), jnp.float32),
            jax.ShapeDtypeStruct((B * N, 64), jnp.float32),
        ],
    )(ft, pt8, wf1t, bf1, wf2t, bf2, wf3t, bf3, wp1t8, bp1)


def _cmat_body(ctr_ref, wp1_ref, c_ref):
    c_ref[...] = jnp.dot(ctr_ref[...], wp1_ref[...], preferred_element_type=jnp.float32)


def _cmat(ctr8, wp1t8):
    return pl.pallas_call(
        _cmat_body,
        out_shape=jax.ShapeDtypeStruct((ROWS, 64), jnp.float32),
    )(ctr8, wp1t8)


def _gath_body(gidx_hbm, a_hbm, p_hbm, c_hbm, h1_hbm, ff_hbm,
               idx0, idx1, ab0, ab1, pb0, pb1, cb0, cb1, h10, h11, fb0, fb1,
               si0, si1, sa0, sa1, sp0, sp1, sc0, sc1, sw0, sw1, sw2, sw3):
    wid = lax.axis_index("s") * NC + lax.axis_index("c")
    base = wid * RPW
    neg_inf = jnp.full((16,), -jnp.inf, jnp.float32)
    bufs = ((idx0, ab0, pb0, cb0, h10, fb0, sa0, sp0, sc0, sw0, sw1),
            (idx1, ab1, pb1, cb1, h11, fb1, sa1, sp1, sc1, sw2, sw3))

    def pair_body(it, _):
        g = it * 2
        cpi = []
        for par in range(2):
            row = base + g + par
            cpi.append(pltpu.async_copy(
                gidx_hbm.at[pl.ds(row * GROUP, GROUP)], bufs[par][0],
                (si0, si1)[par]))
        gathers = []
        for par in range(2):
            row = base + g + par
            idxb, abuf, pbuf, cbuf = bufs[par][:4]
            sa, sp, sc = bufs[par][6:9]
            cpi[par].wait()
            boffv = jnp.full((16,), (row >> 10) * N, jnp.int32)
            for q in range(4):
                idxb[pl.ds(q * 16, 16)] = idxb[pl.ds(q * 16, 16)] + boffv
            gathers.append((
                pltpu.async_copy(a_hbm.at[idxb], abuf, sa),
                pltpu.async_copy(p_hbm.at[idxb], pbuf, sp),
                pltpu.async_copy(c_hbm.at[pl.ds(row * 64, 64)], cbuf, sc)))
        writes = []
        for par in range(2):
            row = base + g + par
            idxb, abuf, pbuf, cbuf, h1buf, ffbuf = bufs[par][:6]
            swh, swf = bufs[par][9:11]
            for cp in gathers[par]:
                cp.wait()

            def rbody(r, acc):
                out = []
                for q in range(4):
                    a = abuf[r, pl.ds(q * 16, 16)]
                    c = cbuf[pl.ds(q * 16, 16)]
                    h1buf[pl.ds(r * 64 + q * 16, 16)] = jnp.maximum(a - c, 0.0)
                    out.append(jnp.maximum(acc[q], pbuf[r, pl.ds(q * 16, 16)]))
                return tuple(out)

            acc = lax.fori_loop(0, GROUP, rbody, (neg_inf,) * 4)
            for q in range(4):
                ffbuf[pl.ds(q * 16, 16)] = acc[q]
            writes.append(pltpu.async_copy(
                h1buf, h1_hbm.at[pl.ds(row * GROUP * 64, GROUP * 64)], swh))
            writes.append(pltpu.async_copy(
                ffbuf, ff_hbm.at[pl.ds(row * 64, 64)], swf))
        for cp in writes:
            cp.wait()
        return 0

    lax.fori_loop(0, RPW // 2, pair_body, 0)


def _gather_l1(gidx, a_rows, p_rows, c_rows):
    f = pl.kernel(
        _gath_body,
        out_type=[
            jax.ShapeDtypeStruct((ROWS * GROUP * 64,), jnp.float32),
            jax.ShapeDtypeStruct((ROWS * 64,), jnp.float32),
        ],
        mesh=plsc.VectorSubcoreMesh(core_axis_name="c", subcore_axis_name="s"),
        compiler_params=pltpu.CompilerParams(needs_layout_passes=False,
                                             use_tc_tiling_on_sc=False),
        scratch_types=(
            [pltpu.VMEM((GROUP,), jnp.int32)] * 2
            + [pltpu.VMEM((GROUP, 64), jnp.float32)] * 4
            + [pltpu.VMEM((64,), jnp.float32)] * 2
            + [pltpu.VMEM((GROUP * 64,), jnp.float32)] * 2
            + [pltpu.VMEM((64,), jnp.float32)] * 2
            + [pltpu.SemaphoreType.DMA] * 12
        ),
    )
    return f(gidx, a_rows, p_rows, c_rows.reshape(ROWS * 64))


_GT = 64


def _mlp2_body(h1_ref, wp2, bp2, wp3, bp3, out_ref):
    h = jnp.maximum(jnp.dot(h1_ref[...], wp2[...], preferred_element_type=jnp.float32) + bp2[...], 0.0)
    h = jnp.dot(h, wp3[...], preferred_element_type=jnp.float32) + bp3[...]
    out_ref[...] = jnp.max(h.reshape(_GT, GROUP, 64), axis=1)


def _mlp2(h1, wp2t, bp2, wp3t, bp3):
    grid = (ROWS // _GT,)
    wspec = pl.BlockSpec((64, 64), lambda i: (0, 0))
    bspec = pl.BlockSpec((1, 64), lambda i: (0, 0))
    return pl.pallas_call(
        _mlp2_body,
        grid=grid,
        in_specs=[
            pl.BlockSpec((_GT * GROUP, 64), lambda i: (i, 0)),
            wspec, bspec, wspec, bspec,
        ],
        out_specs=pl.BlockSpec((_GT, 64), lambda i: (i, 0)),
        out_shape=jax.ShapeDtypeStruct((ROWS, 64), jnp.float32),
    )(h1, wp2t, bp2, wp3t, bp3)


def kernel(points, point_features, Wp1, bp1, Wp2, bp2, Wp3, bp3,
           Wf1, bf1, Wf2, bf2, Wf3, bf3):
    ctrx, ctry, ctrz = _fps(points)
    cx, cy, cz = ctrx.T, ctry.T, ctrz.T
    d2 = _d2(points, cx[:, None, :], cy[:, None, :], cz[:, None, :])
    gidx = _select(d2)

    ft = point_features.transpose(0, 2, 1).reshape(B * N, CIN)
    pt = points.transpose(0, 2, 1).reshape(B * N, 3)
    pt8 = jnp.pad(pt, ((0, 0), (0, 5)))
    wp1t8 = jnp.pad(Wp1.T, ((0, 5), (0, 0)))
    p_rows, a_rows = _precompute(
        ft, pt8, Wf1.T, bf1[None, :], Wf2.T, bf2[None, :], Wf3.T, bf3[None, :],
        wp1t8, bp1[None, :])

    ctr_bk3 = jnp.stack([cx, cy, cz], axis=-1)
    ctr8 = jnp.pad(ctr_bk3.reshape(ROWS, 3), ((0, 0), (0, 5)))
    c_rows = _cmat(ctr8, wp1t8)

    h1, ff = _gather_l1(gidx, a_rows, p_rows, c_rows)
    pp = _mlp2(h1.reshape(ROWS * GROUP, 64), Wp2.T, bp2[None, :], Wp3.T, bp3[None, :])

    ff = ff.reshape(B, K, 64).transpose(0, 2, 1)
    pp = pp.reshape(B, K, 64).transpose(0, 2, 1)
    centroid_features = jnp.concatenate([ff, pp], axis=1)
    centroids = ctr_bk3.reshape(B, 3, K)
    return centroids, centroid_features
